# baseline (speedup 1.0000x reference)
import jax
import jax.numpy as jnp
from jax import lax
from jax.experimental import pallas as pl
from jax.experimental.pallas import tpu as pltpu
from jax.experimental.pallas import tpu_sc as plsc

N = 10000
F = 128
E = 320000
NP = 10240
NC, NS, L = 2, 16, 16
NW = NC * NS
CH = 128
EPT = 10240
EP = NW * EPT
NCHUNK = EPT // CH
PACK = 16384
STRIPE = NP // NS
NEG = 5 * N
POS_CHUNKS = NCHUNK
NEG_CHUNKS = 16
LCHUNKS = POS_CHUNKS + NEG_CHUNKS
NEGP = NW * NEG_CHUNKS * CH

_mesh = lambda: plsc.VectorSubcoreMesh(
    core_axis_name="c", subcore_axis_name="s", num_cores=NC, num_subcores=NS)


def _sc_deg_body(colc_hbm, out_hbm, idx_v, ones_v, z_v, hist_sh):
    cid = lax.axis_index("c")
    sid = lax.axis_index("s")
    wid = cid * NS + sid
    pltpu.sync_copy(colc_hbm.at[pl.ds(wid * NCHUNK, NCHUNK)], idx_v)
    zeros16 = jnp.zeros((L,), jnp.float32)
    ones16 = jnp.ones((L,), jnp.float32)

    def zb(i, c):
        z_v[pl.ds(i * L, L)] = zeros16
        return c
    lax.fori_loop(0, STRIPE // L, zb, 0)

    def ob(i, c):
        ones_v[pl.ds(i * L, L)] = ones16
        return c
    lax.fori_loop(0, CH // L, ob, 0)

    pltpu.sync_copy(z_v, hist_sh.at[pl.ds(sid * STRIPE, STRIPE)])
    plsc.subcore_barrier()

    def sc(j, c):
        pltpu.sync_copy(ones_v, hist_sh.at[idx_v.at[j]], add=True)
        return c
    lax.fori_loop(0, NCHUNK, sc, 0)
    plsc.subcore_barrier()
    pltpu.sync_copy(hist_sh.at[pl.ds(sid * STRIPE, STRIPE)],
                    out_hbm.at[cid, pl.ds(sid * STRIPE, STRIPE)])


_sc_deg = pl.kernel(
    _sc_deg_body,
    out_type=jax.ShapeDtypeStruct((NC, NP), jnp.float32),
    mesh=_mesh(),
    scratch_types=[
        pltpu.VMEM((NCHUNK, CH), jnp.int32),
        pltpu.VMEM((CH,), jnp.float32),
        pltpu.VMEM((STRIPE,), jnp.float32),
        pltpu.VMEM_SHARED((NP,), jnp.float32),
    ],
)


HC = 64


def _sc_push_body(su_hbm, pc_hbm, out_hbm,
                  pidx_v, ridx_v, cidx_v, rows_v, acc_sh,
                  gs0, gs1):
    cid = lax.axis_index("c")
    sid = lax.axis_index("s")
    wid = cid * NS + sid
    gsem = [gs0, gs1]
    pltpu.sync_copy(pc_hbm.at[pl.ds(wid * NCHUNK, NCHUNK)], pidx_v)
    pltpu.sync_copy(su_hbm.at[pl.ds(sid * STRIPE, STRIPE)],
                    acc_sh.at[pl.ds(sid * STRIPE, STRIPE)])
    plsc.subcore_barrier()

    def unpack(j, t):
        for g in range(CH // L):
            v = pidx_v[j, pl.ds(g * L, L)]
            ridx_v[t, pl.ds(g * L, L)] = lax.shift_right_logical(v, 14)
            cidx_v[t, pl.ds(g * L, L)] = lax.bitwise_and(v, PACK - 1)

    def gstart(t):
        pltpu.async_copy(su_hbm.at[ridx_v.at[t]], rows_v.at[t], gsem[t])

    def gwait(t):
        pltpu.make_async_copy(su_hbm.at[ridx_v.at[t]], rows_v.at[t],
                              gsem[t]).wait()

    unpack(0, 0)
    gstart(0)

    def body(k, c):
        for t in range(2):
            gwait(t)

            @pl.when(2 * k + t + 1 < NCHUNK)
            def _():
                unpack(2 * k + t + 1, 1 - t)
                gstart(1 - t)
            pltpu.sync_copy(rows_v.at[t], acc_sh.at[cidx_v.at[t]], add=True)
        return c
    lax.fori_loop(0, NCHUNK // 2, body, 0)
    plsc.subcore_barrier()
    pltpu.sync_copy(acc_sh.at[pl.ds(sid * STRIPE, STRIPE)],
                    out_hbm.at[cid, pl.ds(sid * STRIPE, STRIPE)])


_sc_push = pl.kernel(
    _sc_push_body,
    out_type=jax.ShapeDtypeStruct((NC, NP, F), jnp.float32),
    mesh=_mesh(),
    scratch_types=[
        pltpu.VMEM((NCHUNK, CH), jnp.int32),
        pltpu.VMEM((2, CH), jnp.int32),
        pltpu.VMEM((2, CH), jnp.int32),
        pltpu.VMEM((2, CH, F), jnp.float32),
        pltpu.VMEM_SHARED((NP, F), jnp.float32),
        pltpu.SemaphoreType.DMA,
        pltpu.SemaphoreType.DMA,
    ],
)


NH = LCHUNKS * 2


def _sc_loss_body(rep_hbm, pp_hbm, out_hbm,
                  pidx_v, i0a_v, i1a_v, i0b_v, i1b_v,
                  rows0_v, rows1_v, acc_v, rep_sh,
                  g0A, g1A, g0B, g1B):
    cid = lax.axis_index("c")
    sid = lax.axis_index("s")
    wid = cid * NS + sid
    pltpu.sync_copy(pp_hbm.at[pl.ds(wid * LCHUNKS, LCHUNKS)], pidx_v)
    pltpu.sync_copy(rep_hbm.at[pl.ds(sid * STRIPE, STRIPE)],
                    rep_sh.at[pl.ds(sid * STRIPE, STRIPE)])
    plsc.subcore_barrier()
    lanes = lax.iota(jnp.int32, L)
    perms = [(lanes + k) % L for k in (8, 4, 2, 1)]

    def pv(h, g):
        j = lax.shift_right_logical(h, 1)
        base = lax.bitwise_and(h, 1) * HC
        return pidx_v[j, pl.ds(base + g * L, L)]

    def unpack(h, i0buf, i1buf):
        for g in range(HC // L):
            v = pv(h, g)
            i0buf[pl.ds(g * L, L)] = lax.shift_right_logical(v, 14)
            i1buf[pl.ds(g * L, L)] = lax.bitwise_and(v, PACK - 1)

    def start(h, b):
        if b == 0:
            unpack(h, i0a_v, i1a_v)
            pltpu.async_copy(rep_sh.at[i0a_v], rows0_v.at[0], g0A)
            pltpu.async_copy(rep_sh.at[i1a_v], rows1_v.at[0], g1A)
        else:
            unpack(h, i0b_v, i1b_v)
            pltpu.async_copy(rep_sh.at[i0b_v], rows0_v.at[1], g0B)
            pltpu.async_copy(rep_sh.at[i1b_v], rows1_v.at[1], g1B)

    def wait(b):
        if b == 0:
            pltpu.make_async_copy(rep_sh.at[i0a_v], rows0_v.at[0], g0A).wait()
            pltpu.make_async_copy(rep_sh.at[i1a_v], rows1_v.at[0], g1A).wait()
        else:
            pltpu.make_async_copy(rep_sh.at[i0b_v], rows0_v.at[1], g0B).wait()
            pltpu.make_async_copy(rep_sh.at[i1b_v], rows1_v.at[1], g1B).wait()

    def hsum(p):
        for pm in perms:
            p = p + jnp.take(p, pm)
        return p

    def chunk(h, b, carry):
        loss_a, cnt_a = carry
        tgt_s = jnp.where(h < 2 * POS_CHUNKS, 1.0, 0.0)
        tgt = jnp.full((L,), tgt_s, jnp.float32)
        rb0 = rows0_v.at[b]
        rb1 = rows1_v.at[b]
        for g in range(HC // L):
            v = pv(h, g)
            i0 = lax.shift_right_logical(v, 14)
            i1 = lax.bitwise_and(v, PACK - 1)
            mf = jnp.where(i0 < i1, 1.0, 0.0)

            def ebody(u2, dv):
                for du in range(2):
                    u = 2 * u2 + du
                    e = g * L + u
                    p = jnp.zeros((L,), jnp.float32)
                    for k in range(F // L):
                        p = p + (rb0[e, pl.ds(k * L, L)]
                                 * rb1[e, pl.ds(k * L, L)])
                    s = hsum(p)
                    dv = jnp.where(lanes == u, s, dv)
                return dv
            dot = lax.fori_loop(0, L // 2, ebody,
                                jnp.zeros((L,), jnp.float32))
            d = dot - tgt
            loss_a = loss_a + d * d * mf
            cnt_a = cnt_a + mf
        return loss_a, cnt_a

    start(0, 0)

    def body(k, carry):
        hA = 2 * k
        hB = hA + 1
        start(hB, 1)
        wait(0)
        carry = chunk(hA, 0, carry)

        @pl.when(k < NH // 2 - 1)
        def _():
            start(hB + 1, 0)

        wait(1)
        carry = chunk(hB, 1, carry)
        return carry

    z = jnp.zeros((L,), jnp.float32)
    loss_a, cnt_a = lax.fori_loop(0, NH // 2, body, (z, z))
    acc_v[0, :] = loss_a
    acc_v[1, :] = cnt_a
    pltpu.sync_copy(acc_v, out_hbm.at[wid])


_sc_loss = pl.kernel(
    _sc_loss_body,
    out_type=jax.ShapeDtypeStruct((NW, 2, L), jnp.float32),
    mesh=_mesh(),
    scratch_types=[
        pltpu.VMEM((LCHUNKS, CH), jnp.int32),
        pltpu.VMEM((HC,), jnp.int32),
        pltpu.VMEM((HC,), jnp.int32),
        pltpu.VMEM((HC,), jnp.int32),
        pltpu.VMEM((HC,), jnp.int32),
        pltpu.VMEM((2, HC, F), jnp.float32),
        pltpu.VMEM((2, HC, F), jnp.float32),
        pltpu.VMEM((2, L), jnp.float32),
        pltpu.VMEM_SHARED((NP, F), jnp.float32),
        pltpu.SemaphoreType.DMA,
        pltpu.SemaphoreType.DMA,
        pltpu.SemaphoreType.DMA,
        pltpu.SemaphoreType.DMA,
    ],
)


BLK = 1024


def _tc1_body(x_ref, w_ref, d0_ref, d1_ref, o_ref):
    dinv = lax.rsqrt(d0_ref[...] + d1_ref[...] + 1.0)
    o_ref[...] = jnp.dot(x_ref[...], w_ref[...],
                         preferred_element_type=jnp.float32) * dinv


_tc1 = pl.pallas_call(
    _tc1_body,
    grid=(NP // BLK,),
    in_specs=[
        pl.BlockSpec((BLK, F), lambda i: (i, 0)),
        pl.BlockSpec((F, F), lambda i: (0, 0)),
        pl.BlockSpec((BLK, 1), lambda i: (i, 0)),
        pl.BlockSpec((BLK, 1), lambda i: (i, 0)),
    ],
    out_specs=pl.BlockSpec((BLK, F), lambda i: (i, 0)),
    out_shape=jax.ShapeDtypeStruct((NP, F), jnp.float32),
)


def _tc2_body(a0_ref, a1_ref, su_ref, d0_ref, d1_ref, b1_ref, w_ref, o_ref):
    dinv = lax.rsqrt(d0_ref[...] + d1_ref[...] + 1.0)
    base = (a0_ref[...] + a1_ref[...] - su_ref[...]) * dinv
    h = jnp.maximum(base + b1_ref[...], 0.0)
    o_ref[...] = jnp.dot(h, w_ref[...],
                         preferred_element_type=jnp.float32) * dinv


_tc2 = pl.pallas_call(
    _tc2_body,
    grid=(NP // BLK,),
    in_specs=[
        pl.BlockSpec((BLK, F), lambda i: (i, 0)),
        pl.BlockSpec((BLK, F), lambda i: (i, 0)),
        pl.BlockSpec((BLK, F), lambda i: (i, 0)),
        pl.BlockSpec((BLK, 1), lambda i: (i, 0)),
        pl.BlockSpec((BLK, 1), lambda i: (i, 0)),
        pl.BlockSpec((1, F), lambda i: (0, 0)),
        pl.BlockSpec((F, F), lambda i: (0, 0)),
    ],
    out_specs=pl.BlockSpec((BLK, F), lambda i: (i, 0)),
    out_shape=jax.ShapeDtypeStruct((NP, F), jnp.float32),
)


def _tc3_body(a0_ref, a1_ref, su_ref, d0_ref, d1_ref, b2_ref, o_ref):
    dinv = lax.rsqrt(d0_ref[...] + d1_ref[...] + 1.0)
    o_ref[...] = (a0_ref[...] + a1_ref[...] - su_ref[...]) * dinv + b2_ref[...]


_tc3 = pl.pallas_call(
    _tc3_body,
    grid=(NP // BLK,),
    in_specs=[
        pl.BlockSpec((BLK, F), lambda i: (i, 0)),
        pl.BlockSpec((BLK, F), lambda i: (i, 0)),
        pl.BlockSpec((BLK, F), lambda i: (i, 0)),
        pl.BlockSpec((BLK, 1), lambda i: (i, 0)),
        pl.BlockSpec((BLK, 1), lambda i: (i, 0)),
        pl.BlockSpec((1, F), lambda i: (0, 0)),
    ],
    out_specs=pl.BlockSpec((BLK, F), lambda i: (i, 0)),
    out_shape=jax.ShapeDtypeStruct((NP, F), jnp.float32),
)


def kernel(edge_index, features, W1, b1, W2, b2):
    ei = edge_index
    ar = jnp.arange(EP - E, dtype=jnp.int32)
    row_p = jnp.concatenate([ei[0], (ar * 37) % N])
    col_p = jnp.concatenate([ei[1], N + (ar % (NP - N))])
    pc = (row_p * PACK + col_p).reshape(EP // CH, CH)
    colc_deg = col_p.reshape(EP // CH, CH)
    x_p = jnp.pad(features, ((0, NP - N), (0, 0)))

    deg_parts = _sc_deg(colc_deg)
    d0 = deg_parts[0].reshape(NP, 1)
    d1 = deg_parts[1].reshape(NP, 1)

    su1 = _tc1(x_p, W1, d0, d1)
    b1r = b1.reshape(1, F)
    b2r = b2.reshape(1, F)

    acc1 = _sc_push(su1, pc)
    su2 = _tc2(acc1[0], acc1[1], su1, d0, d1, b1r, W2)
    acc2 = _sc_push(su2, pc)
    rep_p = _tc3(acc2[0], acc2[1], su2, d0, d1, b2r)

    neg = jax.random.randint(jax.random.key(42), (2, NEG), 0, N,
                             dtype=jnp.int32)
    a2 = jnp.arange(EP - E, dtype=jnp.int32)
    pos0 = jnp.concatenate([ei[0], (N // 2) + (a2 % (N // 2))])
    pos1 = jnp.concatenate([ei[1], a2 % (N // 2)])
    a3 = jnp.arange(NEGP - NEG, dtype=jnp.int32)
    neg0 = jnp.concatenate([neg[0], (N // 2) + (a3 % (N // 2))])
    neg1 = jnp.concatenate([neg[1], a3 % (N // 2)])
    pos_pk = pos0 * PACK + pos1
    neg_pk = neg0 * PACK + neg1
    pp = jnp.concatenate([pos_pk.reshape(NW, POS_CHUNKS, CH),
                          neg_pk.reshape(NW, NEG_CHUNKS, CH)],
                         axis=1).reshape(NW * LCHUNKS, CH)

    parts = _sc_loss(rep_p, pp)
    loss_sum = jnp.sum(parts[:, 0, :])
    cnt = jnp.sum(parts[:, 1, :])
    rec_loss = loss_sum * N / cnt
    return rep_p[:N], rec_loss


# speedup vs baseline: 1.0186x; 1.0186x over previous
import numpy as np

import jax
import jax.numpy as jnp
from jax import lax
from jax.experimental import pallas as pl
from jax.experimental.pallas import tpu as pltpu
from jax.experimental.pallas import tpu_sc as plsc

N = 10000
F = 128
E = 320000
NP = 10240
NC, NS, L = 2, 16, 16
NW = NC * NS
CH = 128
EC = E // CH
FULL_CHUNKS = 80
LAST_CHUNKS = EC - (NW - 1) * FULL_CHUNKS
STRIPE = NP // NS
PACK = 16384
HC = 64
NEG = 5 * N
NEGC_PT = 16
NEGP = NW * NEGC_PT * CH

_mesh = lambda: plsc.VectorSubcoreMesh(
    core_axis_name="c", subcore_axis_name="s", num_cores=NC, num_subcores=NS)


def _tf2x32(k0, k1, c0, c1):
    rot1 = (13, 15, 26, 6)
    rot2 = (17, 29, 16, 24)
    ks = [np.uint32(k0), np.uint32(k1),
          np.uint32(k0) ^ np.uint32(k1) ^ np.uint32(0x1BD11BDA)]
    x0 = (c0 + ks[0]).astype(np.uint32)
    x1 = (c1 + ks[1]).astype(np.uint32)

    def rotl(x, d):
        return ((x << np.uint32(d)) | (x >> np.uint32(32 - d))).astype(np.uint32)

    for r in range(5):
        rots = rot1 if r % 2 == 0 else rot2
        for i in range(4):
            x0 = (x0 + x1).astype(np.uint32)
            x1 = rotl(x1, rots[i])
            x1 = (x1 ^ x0).astype(np.uint32)
        x0 = (x0 + ks[(r + 1) % 3]).astype(np.uint32)
        x1 = (x1 + ks[(r + 2) % 3] + np.uint32(r + 1)).astype(np.uint32)
    return x0, x1


def _tf_bits(k, n):
    idx = np.arange(n, dtype=np.uint32)
    y0, y1 = _tf2x32(k[0], k[1], np.zeros_like(idx), idx)
    return (y0 ^ y1).astype(np.uint32)


def _neg_pairs():
    idx = np.arange(2, dtype=np.uint32)
    y0, y1 = _tf2x32(0, 42, np.zeros_like(idx), idx)
    k1, k2 = (y0[0], y1[0]), (y0[1], y1[1])
    n = 2 * NEG
    hi = _tf_bits(k1, n)
    lo = _tf_bits(k2, n)
    span = np.uint32(N)
    mult = np.uint32((np.uint64(65536) % span) ** 2 % span)
    off = ((hi % span) * mult + lo % span) % span
    neg = off.astype(np.int32).reshape(2, NEG)
    padn = NEGP - NEG
    ar = np.arange(padn, dtype=np.int32)
    n0 = np.concatenate([neg[0], (N // 2) + (ar % (N // 2))])
    n1 = np.concatenate([neg[1], ar % (N // 2)])
    return (n0 * PACK + n1).astype(np.int32).reshape(NEGP // CH, CH)


_NEGC = _neg_pairs()


def _sc_deg_body(pc_hbm, out_hbm, pidx_v, colb_v, z_v, hist_sh):
    cid = lax.axis_index("c")
    sid = lax.axis_index("s")
    wid = cid * NS + sid
    nchunks = jnp.where(wid == NW - 1, LAST_CHUNKS, FULL_CHUNKS)

    @pl.when(wid < NW - 1)
    def _():
        pltpu.sync_copy(pc_hbm.at[pl.ds(wid * FULL_CHUNKS, FULL_CHUNKS)],
                        pidx_v)

    @pl.when(wid == NW - 1)
    def _():
        pltpu.sync_copy(pc_hbm.at[pl.ds((NW - 1) * FULL_CHUNKS, LAST_CHUNKS)],
                        pidx_v.at[pl.ds(0, LAST_CHUNKS)])

    zeros16 = jnp.zeros((L,), jnp.float32)
    ones16 = jnp.ones((L,), jnp.float32)

    def zb(i, c):
        z_v[pl.ds(i * L, L)] = zeros16
        return c
    lax.fori_loop(0, STRIPE // L, zb, 0)
    pltpu.sync_copy(z_v, hist_sh.at[pl.ds(sid * STRIPE, STRIPE)])
    plsc.subcore_barrier()

    ones_v = z_v

    def ob(i, c):
        ones_v[pl.ds(i * L, L)] = ones16
        return c
    lax.fori_loop(0, CH // L, ob, 0)

    def sc(j, c):
        for g in range(CH // L):
            v = pidx_v[j, pl.ds(g * L, L)]
            colb_v[pl.ds(g * L, L)] = lax.bitwise_and(v, PACK - 1)
        pltpu.sync_copy(ones_v.at[pl.ds(0, CH)], hist_sh.at[colb_v], add=True)
        return c
    lax.fori_loop(0, nchunks, sc, 0)
    plsc.subcore_barrier()
    pltpu.sync_copy(hist_sh.at[pl.ds(sid * STRIPE, STRIPE)],
                    out_hbm.at[cid, pl.ds(sid * STRIPE, STRIPE)])


_sc_deg = pl.kernel(
    _sc_deg_body,
    out_type=jax.ShapeDtypeStruct((NC, NP), jnp.float32),
    mesh=_mesh(),
    scratch_types=[
        pltpu.VMEM((FULL_CHUNKS, CH), jnp.int32),
        pltpu.VMEM((CH,), jnp.int32),
        pltpu.VMEM((STRIPE,), jnp.float32),
        pltpu.VMEM_SHARED((NP,), jnp.float32),
    ],
)


def _su_stripe_init(su_hbm, dst_sh, sid):
    @pl.when(sid < NS - 1)
    def _():
        pltpu.sync_copy(su_hbm.at[pl.ds(sid * STRIPE, STRIPE)],
                        dst_sh.at[pl.ds(sid * STRIPE, STRIPE)])

    @pl.when(sid == NS - 1)
    def _():
        pltpu.sync_copy(su_hbm.at[pl.ds((NS - 1) * STRIPE, N - (NS - 1) * STRIPE)],
                        dst_sh.at[pl.ds((NS - 1) * STRIPE, N - (NS - 1) * STRIPE)])


def _sc_push_body(su_hbm, pc_hbm, out_hbm,
                  pidx_v, ridx_v, cidx_v, rows_v, acc_sh, gs0, gs1):
    cid = lax.axis_index("c")
    sid = lax.axis_index("s")
    wid = cid * NS + sid
    gsem = [gs0, gs1]
    nchunks = jnp.where(wid == NW - 1, LAST_CHUNKS, FULL_CHUNKS)

    @pl.when(wid < NW - 1)
    def _():
        pltpu.sync_copy(pc_hbm.at[pl.ds(wid * FULL_CHUNKS, FULL_CHUNKS)],
                        pidx_v)

    @pl.when(wid == NW - 1)
    def _():
        pltpu.sync_copy(pc_hbm.at[pl.ds((NW - 1) * FULL_CHUNKS, LAST_CHUNKS)],
                        pidx_v.at[pl.ds(0, LAST_CHUNKS)])

    _su_stripe_init(su_hbm, acc_sh, sid)
    plsc.subcore_barrier()

    def unpack(j, t):
        for g in range(CH // L):
            v = pidx_v[j, pl.ds(g * L, L)]
            ridx_v[t, pl.ds(g * L, L)] = lax.shift_right_logical(v, 14)
            cidx_v[t, pl.ds(g * L, L)] = lax.bitwise_and(v, PACK - 1)

    def gstart(t):
        pltpu.async_copy(su_hbm.at[ridx_v.at[t]], rows_v.at[t], gsem[t])

    def gwait(t):
        pltpu.make_async_copy(su_hbm.at[ridx_v.at[t]], rows_v.at[t],
                              gsem[t]).wait()

    unpack(0, 0)
    gstart(0)

    def body(k, c):
        for t in range(2):
            gwait(t)

            @pl.when(2 * k + t + 1 < nchunks)
            def _():
                unpack(2 * k + t + 1, 1 - t)
                gstart(1 - t)
            pltpu.sync_copy(rows_v.at[t], acc_sh.at[cidx_v.at[t]], add=True)
        return c
    lax.fori_loop(0, nchunks // 2, body, 0)
    plsc.subcore_barrier()
    pltpu.sync_copy(acc_sh.at[pl.ds(sid * STRIPE, STRIPE)],
                    out_hbm.at[cid, pl.ds(sid * STRIPE, STRIPE)])


_sc_push = pl.kernel(
    _sc_push_body,
    out_type=jax.ShapeDtypeStruct((NC, NP, F), jnp.float32),
    mesh=_mesh(),
    scratch_types=[
        pltpu.VMEM((FULL_CHUNKS, CH), jnp.int32),
        pltpu.VMEM((2, CH), jnp.int32),
        pltpu.VMEM((2, CH), jnp.int32),
        pltpu.VMEM((2, CH, F), jnp.float32),
        pltpu.VMEM_SHARED((NP, F), jnp.float32),
        pltpu.SemaphoreType.DMA,
        pltpu.SemaphoreType.DMA,
    ],
)


def _sc_loss_body(rep_hbm, pc_hbm, ng_hbm, out_hbm,
                  pidx_v, nidx_v, i0a_v, i1a_v, i0b_v, i1b_v,
                  rows0_v, rows1_v, acc_v, rep_sh,
                  g0A, g1A, g0B, g1B):
    cid = lax.axis_index("c")
    sid = lax.axis_index("s")
    wid = cid * NS + sid
    npos_h = jnp.where(wid == NW - 1, 2 * LAST_CHUNKS, 2 * FULL_CHUNKS)

    @pl.when(wid < NW - 1)
    def _():
        pltpu.sync_copy(pc_hbm.at[pl.ds(wid * FULL_CHUNKS, FULL_CHUNKS)],
                        pidx_v)

    @pl.when(wid == NW - 1)
    def _():
        pltpu.sync_copy(pc_hbm.at[pl.ds((NW - 1) * FULL_CHUNKS, LAST_CHUNKS)],
                        pidx_v.at[pl.ds(0, LAST_CHUNKS)])

    pltpu.sync_copy(ng_hbm.at[pl.ds(wid * NEGC_PT, NEGC_PT)], nidx_v)
    _su_stripe_init(rep_hbm, rep_sh, sid)
    plsc.subcore_barrier()
    lanes = lax.iota(jnp.int32, L)
    perms = [(lanes + k) % L for k in (8, 4, 2, 1)]

    def hsum(p):
        for pm in perms:
            p = p + jnp.take(p, pm)
        return p

    def mk_pv(idx_ref):
        def pv(h, g):
            j = lax.shift_right_logical(h, 1)
            base = lax.bitwise_and(h, 1) * HC
            return idx_ref[j, pl.ds(base + g * L, L)]
        return pv

    def mk_ops(pv):
        def unpack(h, i0buf, i1buf):
            for g in range(HC // L):
                v = pv(h, g)
                i0buf[pl.ds(g * L, L)] = lax.shift_right_logical(v, 14)
                i1buf[pl.ds(g * L, L)] = lax.bitwise_and(v, PACK - 1)

        def start(h, b):
            if b == 0:
                unpack(h, i0a_v, i1a_v)
                pltpu.async_copy(rep_sh.at[i0a_v], rows0_v.at[0], g0A)
                pltpu.async_copy(rep_sh.at[i1a_v], rows1_v.at[0], g1A)
            else:
                unpack(h, i0b_v, i1b_v)
                pltpu.async_copy(rep_sh.at[i0b_v], rows0_v.at[1], g0B)
                pltpu.async_copy(rep_sh.at[i1b_v], rows1_v.at[1], g1B)

        def wait(b):
            if b == 0:
                pltpu.make_async_copy(rep_sh.at[i0a_v], rows0_v.at[0], g0A).wait()
                pltpu.make_async_copy(rep_sh.at[i1a_v], rows1_v.at[0], g1A).wait()
            else:
                pltpu.make_async_copy(rep_sh.at[i0b_v], rows0_v.at[1], g0B).wait()
                pltpu.make_async_copy(rep_sh.at[i1b_v], rows1_v.at[1], g1B).wait()

        def chunk(h, b, tgt, carry):
            loss_a, cnt_a = carry
            rb0 = rows0_v.at[b]
            rb1 = rows1_v.at[b]
            for g in range(HC // L):
                v = pv(h, g)
                i0 = lax.shift_right_logical(v, 14)
                i1 = lax.bitwise_and(v, PACK - 1)
                mf = jnp.where(i0 < i1, 1.0, 0.0)

                def ebody(u2, dv):
                    for du in range(2):
                        u = 2 * u2 + du
                        e = g * L + u
                        p = jnp.zeros((L,), jnp.float32)
                        for k in range(F // L):
                            p = p + (rb0[e, pl.ds(k * L, L)]
                                     * rb1[e, pl.ds(k * L, L)])
                        s = hsum(p)
                        dv = jnp.where(lanes == u, s, dv)
                    return dv
                dot = lax.fori_loop(0, L // 2, ebody,
                                    jnp.zeros((L,), jnp.float32))
                d = dot - tgt
                loss_a = loss_a + d * d * mf
                cnt_a = cnt_a + mf
            return loss_a, cnt_a

        return start, wait, chunk

    def run_region(ops, nh, tgt_s, carry):
        start, wait, chunk = ops
        tgt = jnp.full((L,), tgt_s, jnp.float32)
        start(0, 0)

        def body(k, carry):
            hA = 2 * k
            hB = hA + 1
            start(hB, 1)
            wait(0)
            carry = chunk(hA, 0, tgt, carry)

            @pl.when(hB + 1 < nh)
            def _():
                start(hB + 1, 0)

            wait(1)
            carry = chunk(hB, 1, tgt, carry)
            return carry
        return lax.fori_loop(0, nh // 2, body, carry)

    z = jnp.zeros((L,), jnp.float32)
    carry = run_region(mk_ops(mk_pv(pidx_v)), npos_h, 1.0, (z, z))
    carry = run_region(mk_ops(mk_pv(nidx_v)), 2 * NEGC_PT, 0.0, carry)
    loss_a, cnt_a = carry
    acc_v[0, :] = loss_a
    acc_v[1, :] = cnt_a
    pltpu.sync_copy(acc_v, out_hbm.at[wid])


_sc_loss = pl.kernel(
    _sc_loss_body,
    out_type=jax.ShapeDtypeStruct((NW, 2, L), jnp.float32),
    mesh=_mesh(),
    scratch_types=[
        pltpu.VMEM((FULL_CHUNKS, CH), jnp.int32),
        pltpu.VMEM((NEGC_PT, CH), jnp.int32),
        pltpu.VMEM((HC,), jnp.int32),
        pltpu.VMEM((HC,), jnp.int32),
        pltpu.VMEM((HC,), jnp.int32),
        pltpu.VMEM((HC,), jnp.int32),
        pltpu.VMEM((2, HC, F), jnp.float32),
        pltpu.VMEM((2, HC, F), jnp.float32),
        pltpu.VMEM((2, L), jnp.float32),
        pltpu.VMEM_SHARED((N, F), jnp.float32),
        pltpu.SemaphoreType.DMA,
        pltpu.SemaphoreType.DMA,
        pltpu.SemaphoreType.DMA,
        pltpu.SemaphoreType.DMA,
    ],
)


BLK = 1000


def _tc1_body(x_ref, w_ref, d0_ref, d1_ref, o_ref):
    dinv = lax.rsqrt(d0_ref[...] + d1_ref[...] + 1.0)
    o_ref[...] = jnp.dot(x_ref[...], w_ref[...],
                         preferred_element_type=jnp.float32) * dinv


_tc1 = pl.pallas_call(
    _tc1_body,
    grid=(N // BLK,),
    in_specs=[
        pl.BlockSpec((BLK, F), lambda i: (i, 0)),
        pl.BlockSpec((F, F), lambda i: (0, 0)),
        pl.BlockSpec((BLK, 1), lambda i: (i, 0)),
        pl.BlockSpec((BLK, 1), lambda i: (i, 0)),
    ],
    out_specs=pl.BlockSpec((BLK, F), lambda i: (i, 0)),
    out_shape=jax.ShapeDtypeStruct((N, F), jnp.float32),
)


def _tc2_body(a0_ref, a1_ref, su_ref, d0_ref, d1_ref, b1_ref, w_ref, o_ref):
    dinv = lax.rsqrt(d0_ref[...] + d1_ref[...] + 1.0)
    base = (a0_ref[0] + a1_ref[0] - su_ref[...]) * dinv
    h = jnp.maximum(base + b1_ref[...], 0.0)
    o_ref[...] = jnp.dot(h, w_ref[...],
                         preferred_element_type=jnp.float32) * dinv


_tc2 = pl.pallas_call(
    _tc2_body,
    grid=(N // BLK,),
    in_specs=[
        pl.BlockSpec((1, BLK, F), lambda i: (0, i, 0)),
        pl.BlockSpec((1, BLK, F), lambda i: (1, i, 0)),
        pl.BlockSpec((BLK, F), lambda i: (i, 0)),
        pl.BlockSpec((BLK, 1), lambda i: (i, 0)),
        pl.BlockSpec((BLK, 1), lambda i: (i, 0)),
        pl.BlockSpec((1, F), lambda i: (0, 0)),
        pl.BlockSpec((F, F), lambda i: (0, 0)),
    ],
    out_specs=pl.BlockSpec((BLK, F), lambda i: (i, 0)),
    out_shape=jax.ShapeDtypeStruct((N, F), jnp.float32),
)


def _tc3_body(a0_ref, a1_ref, su_ref, d0_ref, d1_ref, b2_ref, o_ref):
    dinv = lax.rsqrt(d0_ref[...] + d1_ref[...] + 1.0)
    o_ref[...] = (a0_ref[0] + a1_ref[0] - su_ref[...]) * dinv + b2_ref[...]


_tc3 = pl.pallas_call(
    _tc3_body,
    grid=(N // BLK,),
    in_specs=[
        pl.BlockSpec((1, BLK, F), lambda i: (0, i, 0)),
        pl.BlockSpec((1, BLK, F), lambda i: (1, i, 0)),
        pl.BlockSpec((BLK, F), lambda i: (i, 0)),
        pl.BlockSpec((BLK, 1), lambda i: (i, 0)),
        pl.BlockSpec((BLK, 1), lambda i: (i, 0)),
        pl.BlockSpec((1, F), lambda i: (0, 0)),
    ],
    out_specs=pl.BlockSpec((BLK, F), lambda i: (i, 0)),
    out_shape=jax.ShapeDtypeStruct((N, F), jnp.float32),
)


def kernel(edge_index, features, W1, b1, W2, b2):
    er = edge_index.reshape(2, EC, CH)
    pc = er[0] * PACK + er[1]
    negc = jnp.asarray(_NEGC)

    deg_parts = _sc_deg(pc)
    d0 = deg_parts[0].reshape(NP, 1)
    d1 = deg_parts[1].reshape(NP, 1)
    b1r = b1.reshape(1, F)
    b2r = b2.reshape(1, F)

    su1 = _tc1(features, W1, d0, d1)
    acc1 = _sc_push(su1, pc)
    su2 = _tc2(acc1, acc1, su1, d0, d1, b1r, W2)
    acc2 = _sc_push(su2, pc)
    rep = _tc3(acc2, acc2, su2, d0, d1, b2r)

    parts = _sc_loss(rep, pc, negc)
    loss_sum = jnp.sum(parts[:, 0, :])
    cnt = jnp.sum(parts[:, 1, :])
    rec_loss = loss_sum * N / cnt
    return rep, rec_loss


# speedup vs baseline: 1.0301x; 1.0113x over previous
import numpy as np

import jax
import jax.numpy as jnp
from jax import lax
from jax.experimental import pallas as pl
from jax.experimental.pallas import tpu as pltpu
from jax.experimental.pallas import tpu_sc as plsc

N = 10000
F = 128
E = 320000
NP = 10240
NC, NS, L = 2, 16, 16
NW = NC * NS
CH = 128
EC = E // CH
FULL_CHUNKS = 80
LAST_CHUNKS = EC - (NW - 1) * FULL_CHUNKS
STRIPE = NP // NS
PACK = 16384
HC = 64
NEG = 5 * N
NEGC_PT = 16
NEGP = NW * NEGC_PT * CH

_mesh = lambda: plsc.VectorSubcoreMesh(
    core_axis_name="c", subcore_axis_name="s", num_cores=NC, num_subcores=NS)


def _tf2x32(k0, k1, c0, c1):
    rot1 = (13, 15, 26, 6)
    rot2 = (17, 29, 16, 24)
    ks = [np.uint32(k0), np.uint32(k1),
          np.uint32(k0) ^ np.uint32(k1) ^ np.uint32(0x1BD11BDA)]
    x0 = (c0 + ks[0]).astype(np.uint32)
    x1 = (c1 + ks[1]).astype(np.uint32)

    def rotl(x, d):
        return ((x << np.uint32(d)) | (x >> np.uint32(32 - d))).astype(np.uint32)

    for r in range(5):
        rots = rot1 if r % 2 == 0 else rot2
        for i in range(4):
            x0 = (x0 + x1).astype(np.uint32)
            x1 = rotl(x1, rots[i])
            x1 = (x1 ^ x0).astype(np.uint32)
        x0 = (x0 + ks[(r + 1) % 3]).astype(np.uint32)
        x1 = (x1 + ks[(r + 2) % 3] + np.uint32(r + 1)).astype(np.uint32)
    return x0, x1


def _tf_bits(k, n):
    idx = np.arange(n, dtype=np.uint32)
    y0, y1 = _tf2x32(k[0], k[1], np.zeros_like(idx), idx)
    return (y0 ^ y1).astype(np.uint32)


def _neg_pairs():
    idx = np.arange(2, dtype=np.uint32)
    y0, y1 = _tf2x32(0, 42, np.zeros_like(idx), idx)
    k1, k2 = (y0[0], y1[0]), (y0[1], y1[1])
    n = 2 * NEG
    hi = _tf_bits(k1, n)
    lo = _tf_bits(k2, n)
    span = np.uint32(N)
    mult = np.uint32((np.uint64(65536) % span) ** 2 % span)
    off = ((hi % span) * mult + lo % span) % span
    neg = off.astype(np.int32).reshape(2, NEG)
    padn = NEGP - NEG
    ar = np.arange(padn, dtype=np.int32)
    n0 = np.concatenate([neg[0], (N // 2) + (ar % (N // 2))])
    n1 = np.concatenate([neg[1], ar % (N // 2)])
    return (n0 * PACK + n1).astype(np.int32).reshape(NEGP // CH, CH)


_NEGC = _neg_pairs()


PCROWS = NW * FULL_CHUNKS


def _sc_deg_body(er0_hbm, er1_hbm, out_hbm, pc_hbm, r_v, c_v, z_v, hist_sh):
    cid = lax.axis_index("c")
    sid = lax.axis_index("s")
    wid = cid * NS + sid
    base = wid * FULL_CHUNKS

    @pl.when(wid < NW - 1)
    def _():
        pltpu.sync_copy(er0_hbm.at[pl.ds(base, FULL_CHUNKS)], r_v)
        pltpu.sync_copy(er1_hbm.at[pl.ds(base, FULL_CHUNKS)], c_v)

    @pl.when(wid == NW - 1)
    def _():
        pltpu.sync_copy(er0_hbm.at[pl.ds((NW - 1) * FULL_CHUNKS, LAST_CHUNKS)],
                        r_v.at[pl.ds(0, LAST_CHUNKS)])
        pltpu.sync_copy(er1_hbm.at[pl.ds((NW - 1) * FULL_CHUNKS, LAST_CHUNKS)],
                        c_v.at[pl.ds(0, LAST_CHUNKS)])
        lanes = lax.iota(jnp.int32, L)

        def fb(j, cc):
            for g in range(CH // L):
                x = lanes + g * L
                r_v[j, pl.ds(g * L, L)] = x
                c_v[j, pl.ds(g * L, L)] = N + lax.bitwise_and(x, 127)
            return cc
        lax.fori_loop(LAST_CHUNKS, FULL_CHUNKS, fb, 0)

    zeros16 = jnp.zeros((L,), jnp.float32)
    ones16 = jnp.ones((L,), jnp.float32)

    def zb(i, c):
        z_v[pl.ds(i * L, L)] = zeros16
        return c
    lax.fori_loop(0, STRIPE // L, zb, 0)
    pltpu.sync_copy(z_v, hist_sh.at[pl.ds(sid * STRIPE, STRIPE)])
    plsc.subcore_barrier()

    ones_v = z_v

    def ob(i, c):
        ones_v[pl.ds(i * L, L)] = ones16
        return c
    lax.fori_loop(0, CH // L, ob, 0)

    def sc(j, c):
        pltpu.sync_copy(ones_v.at[pl.ds(0, CH)], hist_sh.at[c_v.at[j]],
                        add=True)
        for g in range(CH // L):
            r = r_v[j, pl.ds(g * L, L)]
            cv = c_v[j, pl.ds(g * L, L)]
            r_v[j, pl.ds(g * L, L)] = r * PACK + cv
        return c
    lax.fori_loop(0, FULL_CHUNKS, sc, 0)
    pltpu.sync_copy(r_v, pc_hbm.at[pl.ds(base, FULL_CHUNKS)])
    plsc.subcore_barrier()
    pltpu.sync_copy(hist_sh.at[pl.ds(sid * STRIPE, STRIPE)],
                    out_hbm.at[cid, pl.ds(sid * STRIPE, STRIPE)])


_sc_deg = pl.kernel(
    _sc_deg_body,
    out_type=(jax.ShapeDtypeStruct((NC, NP), jnp.float32),
              jax.ShapeDtypeStruct((PCROWS, CH), jnp.int32)),
    mesh=_mesh(),
    scratch_types=[
        pltpu.VMEM((FULL_CHUNKS, CH), jnp.int32),
        pltpu.VMEM((FULL_CHUNKS, CH), jnp.int32),
        pltpu.VMEM((STRIPE,), jnp.float32),
        pltpu.VMEM_SHARED((NP,), jnp.float32),
    ],
)


def _su_stripe_init(su_hbm, dst_sh, sid):
    @pl.when(sid < NS - 1)
    def _():
        pltpu.sync_copy(su_hbm.at[pl.ds(sid * STRIPE, STRIPE)],
                        dst_sh.at[pl.ds(sid * STRIPE, STRIPE)])

    @pl.when(sid == NS - 1)
    def _():
        pltpu.sync_copy(su_hbm.at[pl.ds((NS - 1) * STRIPE, N - (NS - 1) * STRIPE)],
                        dst_sh.at[pl.ds((NS - 1) * STRIPE, N - (NS - 1) * STRIPE)])


def _sc_push_body(su_hbm, pc_hbm, out_hbm,
                  pidx_v, ridx_v, cidx_v, rows_v, acc_sh, gs0, gs1):
    cid = lax.axis_index("c")
    sid = lax.axis_index("s")
    wid = cid * NS + sid
    gsem = [gs0, gs1]
    pltpu.sync_copy(pc_hbm.at[pl.ds(wid * FULL_CHUNKS, FULL_CHUNKS)], pidx_v)
    _su_stripe_init(su_hbm, acc_sh, sid)
    plsc.subcore_barrier()

    def unpack(j, t):
        for g in range(CH // L):
            v = pidx_v[j, pl.ds(g * L, L)]
            ridx_v[t, pl.ds(g * L, L)] = lax.shift_right_logical(v, 14)
            cidx_v[t, pl.ds(g * L, L)] = lax.bitwise_and(v, PACK - 1)

    def gstart(t):
        pltpu.async_copy(su_hbm.at[ridx_v.at[t]], rows_v.at[t], gsem[t])

    def gwait(t):
        pltpu.make_async_copy(su_hbm.at[ridx_v.at[t]], rows_v.at[t],
                              gsem[t]).wait()

    unpack(0, 0)
    gstart(0)

    def body(k, c):
        for t in range(2):
            gwait(t)

            @pl.when(2 * k + t + 1 < FULL_CHUNKS)
            def _():
                unpack(2 * k + t + 1, 1 - t)
                gstart(1 - t)
            pltpu.sync_copy(rows_v.at[t], acc_sh.at[cidx_v.at[t]], add=True)
        return c
    lax.fori_loop(0, FULL_CHUNKS // 2, body, 0)
    plsc.subcore_barrier()
    pltpu.sync_copy(acc_sh.at[pl.ds(sid * STRIPE, STRIPE)],
                    out_hbm.at[cid, pl.ds(sid * STRIPE, STRIPE)])


_sc_push = pl.kernel(
    _sc_push_body,
    out_type=jax.ShapeDtypeStruct((NC, NP, F), jnp.float32),
    mesh=_mesh(),
    scratch_types=[
        pltpu.VMEM((FULL_CHUNKS, CH), jnp.int32),
        pltpu.VMEM((2, CH), jnp.int32),
        pltpu.VMEM((2, CH), jnp.int32),
        pltpu.VMEM((2, CH, F), jnp.float32),
        pltpu.VMEM_SHARED((NP, F), jnp.float32),
        pltpu.SemaphoreType.DMA,
        pltpu.SemaphoreType.DMA,
    ],
)


def _sc_loss_body(rep_hbm, pc_hbm, ng_hbm, out_hbm,
                  pidx_v, nidx_v, i0a_v, i1a_v, i0b_v, i1b_v,
                  rows0_v, rows1_v, acc_v, rep_sh,
                  g0A, g1A, g0B, g1B):
    cid = lax.axis_index("c")
    sid = lax.axis_index("s")
    wid = cid * NS + sid
    pltpu.sync_copy(pc_hbm.at[pl.ds(wid * FULL_CHUNKS, FULL_CHUNKS)], pidx_v)
    pltpu.sync_copy(ng_hbm.at[pl.ds(wid * NEGC_PT, NEGC_PT)], nidx_v)
    _su_stripe_init(rep_hbm, rep_sh, sid)
    plsc.subcore_barrier()
    lanes = lax.iota(jnp.int32, L)
    perms = [(lanes + k) % L for k in (8, 4, 2, 1)]

    def hsum(p):
        for pm in perms:
            p = p + jnp.take(p, pm)
        return p

    def mk_pv(idx_ref):
        def pv(h, g):
            j = lax.shift_right_logical(h, 1)
            base = lax.bitwise_and(h, 1) * HC
            return idx_ref[j, pl.ds(base + g * L, L)]
        return pv

    def mk_ops(pv):
        def unpack(h, i0buf, i1buf):
            for g in range(HC // L):
                v = pv(h, g)
                i0buf[pl.ds(g * L, L)] = lax.shift_right_logical(v, 14)
                i1buf[pl.ds(g * L, L)] = lax.bitwise_and(v, PACK - 1)

        def start(h, b):
            if b == 0:
                unpack(h, i0a_v, i1a_v)
                pltpu.async_copy(rep_sh.at[i0a_v], rows0_v.at[0], g0A)
                pltpu.async_copy(rep_sh.at[i1a_v], rows1_v.at[0], g1A)
            else:
                unpack(h, i0b_v, i1b_v)
                pltpu.async_copy(rep_sh.at[i0b_v], rows0_v.at[1], g0B)
                pltpu.async_copy(rep_sh.at[i1b_v], rows1_v.at[1], g1B)

        def wait(b):
            if b == 0:
                pltpu.make_async_copy(rep_sh.at[i0a_v], rows0_v.at[0], g0A).wait()
                pltpu.make_async_copy(rep_sh.at[i1a_v], rows1_v.at[0], g1A).wait()
            else:
                pltpu.make_async_copy(rep_sh.at[i0b_v], rows0_v.at[1], g0B).wait()
                pltpu.make_async_copy(rep_sh.at[i1b_v], rows1_v.at[1], g1B).wait()

        def chunk(h, b, tgt, carry):
            loss_a, cnt_a = carry
            rb0 = rows0_v.at[b]
            rb1 = rows1_v.at[b]
            for g in range(HC // L):
                v = pv(h, g)
                i0 = lax.shift_right_logical(v, 14)
                i1 = lax.bitwise_and(v, PACK - 1)
                msk = (i0 < i1) & (i1 < jnp.int32(N))

                def ebody(u2, dv):
                    for du in range(2):
                        u = 2 * u2 + du
                        e = g * L + u
                        p = jnp.zeros((L,), jnp.float32)
                        for k in range(F // L):
                            p = p + (rb0[e, pl.ds(k * L, L)]
                                     * rb1[e, pl.ds(k * L, L)])
                        s = hsum(p)
                        dv = jnp.where(lanes == u, s, dv)
                    return dv
                dot = lax.fori_loop(0, L // 2, ebody,
                                    jnp.zeros((L,), jnp.float32))
                d = dot - tgt
                loss_a = loss_a + jnp.where(msk, d * d, 0.0)
                cnt_a = cnt_a + jnp.where(msk, 1.0, 0.0)
            return loss_a, cnt_a

        return start, wait, chunk

    def run_region(ops, nh, tgt_s, carry):
        start, wait, chunk = ops
        tgt = jnp.full((L,), tgt_s, jnp.float32)
        start(0, 0)

        def body(k, carry):
            hA = 2 * k
            hB = hA + 1
            start(hB, 1)
            wait(0)
            carry = chunk(hA, 0, tgt, carry)

            @pl.when(hB + 1 < nh)
            def _():
                start(hB + 1, 0)

            wait(1)
            carry = chunk(hB, 1, tgt, carry)
            return carry
        return lax.fori_loop(0, nh // 2, body, carry)

    z = jnp.zeros((L,), jnp.float32)
    carry = run_region(mk_ops(mk_pv(pidx_v)), 2 * FULL_CHUNKS, 1.0, (z, z))
    carry = run_region(mk_ops(mk_pv(nidx_v)), 2 * NEGC_PT, 0.0, carry)
    loss_a, cnt_a = carry
    acc_v[0, :] = loss_a
    acc_v[1, :] = cnt_a
    pltpu.sync_copy(acc_v, out_hbm.at[wid])


_sc_loss = pl.kernel(
    _sc_loss_body,
    out_type=jax.ShapeDtypeStruct((NW, 2, L), jnp.float32),
    mesh=_mesh(),
    scratch_types=[
        pltpu.VMEM((FULL_CHUNKS, CH), jnp.int32),
        pltpu.VMEM((NEGC_PT, CH), jnp.int32),
        pltpu.VMEM((HC,), jnp.int32),
        pltpu.VMEM((HC,), jnp.int32),
        pltpu.VMEM((HC,), jnp.int32),
        pltpu.VMEM((HC,), jnp.int32),
        pltpu.VMEM((2, HC, F), jnp.float32),
        pltpu.VMEM((2, HC, F), jnp.float32),
        pltpu.VMEM((2, L), jnp.float32),
        pltpu.VMEM_SHARED((NP, F), jnp.float32),
        pltpu.SemaphoreType.DMA,
        pltpu.SemaphoreType.DMA,
        pltpu.SemaphoreType.DMA,
        pltpu.SemaphoreType.DMA,
    ],
)


BLK = 2000


def _tc1_body(x_ref, w_ref, d0_ref, d1_ref, o_ref):
    dinv = lax.rsqrt(d0_ref[...] + d1_ref[...] + 1.0)
    o_ref[...] = jnp.dot(x_ref[...], w_ref[...],
                         preferred_element_type=jnp.float32) * dinv


_tc1 = pl.pallas_call(
    _tc1_body,
    grid=(N // BLK,),
    in_specs=[
        pl.BlockSpec((BLK, F), lambda i: (i, 0)),
        pl.BlockSpec((F, F), lambda i: (0, 0)),
        pl.BlockSpec((BLK, 1), lambda i: (i, 0)),
        pl.BlockSpec((BLK, 1), lambda i: (i, 0)),
    ],
    out_specs=pl.BlockSpec((BLK, F), lambda i: (i, 0)),
    out_shape=jax.ShapeDtypeStruct((N, F), jnp.float32),
)


def _tc2_body(a0_ref, a1_ref, su_ref, d0_ref, d1_ref, b1_ref, w_ref, o_ref):
    dinv = lax.rsqrt(d0_ref[...] + d1_ref[...] + 1.0)
    base = (a0_ref[0] + a1_ref[0] - su_ref[...]) * dinv
    h = jnp.maximum(base + b1_ref[...], 0.0)
    o_ref[...] = jnp.dot(h, w_ref[...],
                         preferred_element_type=jnp.float32) * dinv


_tc2 = pl.pallas_call(
    _tc2_body,
    grid=(N // BLK,),
    in_specs=[
        pl.BlockSpec((1, BLK, F), lambda i: (0, i, 0)),
        pl.BlockSpec((1, BLK, F), lambda i: (1, i, 0)),
        pl.BlockSpec((BLK, F), lambda i: (i, 0)),
        pl.BlockSpec((BLK, 1), lambda i: (i, 0)),
        pl.BlockSpec((BLK, 1), lambda i: (i, 0)),
        pl.BlockSpec((1, F), lambda i: (0, 0)),
        pl.BlockSpec((F, F), lambda i: (0, 0)),
    ],
    out_specs=pl.BlockSpec((BLK, F), lambda i: (i, 0)),
    out_shape=jax.ShapeDtypeStruct((N, F), jnp.float32),
)


def _tc3_body(a0_ref, a1_ref, su_ref, d0_ref, d1_ref, b2_ref, o_ref):
    dinv = lax.rsqrt(d0_ref[...] + d1_ref[...] + 1.0)
    o_ref[...] = (a0_ref[0] + a1_ref[0] - su_ref[...]) * dinv + b2_ref[...]


_tc3 = pl.pallas_call(
    _tc3_body,
    grid=(N // BLK,),
    in_specs=[
        pl.BlockSpec((1, BLK, F), lambda i: (0, i, 0)),
        pl.BlockSpec((1, BLK, F), lambda i: (1, i, 0)),
        pl.BlockSpec((BLK, F), lambda i: (i, 0)),
        pl.BlockSpec((BLK, 1), lambda i: (i, 0)),
        pl.BlockSpec((BLK, 1), lambda i: (i, 0)),
        pl.BlockSpec((1, F), lambda i: (0, 0)),
    ],
    out_specs=pl.BlockSpec((BLK, F), lambda i: (i, 0)),
    out_shape=jax.ShapeDtypeStruct((N, F), jnp.float32),
)


def kernel(edge_index, features, W1, b1, W2, b2):
    er = edge_index.reshape(2, EC, CH)
    negc = jnp.asarray(_NEGC)

    deg_parts, pc = _sc_deg(er[0], er[1])
    d0 = deg_parts[0].reshape(NP, 1)
    d1 = deg_parts[1].reshape(NP, 1)
    b1r = b1.reshape(1, F)
    b2r = b2.reshape(1, F)

    su1 = _tc1(features, W1, d0, d1)
    acc1 = _sc_push(su1, pc)
    su2 = _tc2(acc1, acc1, su1, d0, d1, b1r, W2)
    acc2 = _sc_push(su2, pc)
    rep = _tc3(acc2, acc2, su2, d0, d1, b2r)

    parts = _sc_loss(rep, pc, negc)
    loss_sum = jnp.sum(parts[:, 0, :])
    cnt = jnp.sum(parts[:, 1, :])
    rec_loss = loss_sum * N / cnt
    return rep, rec_loss


# speedup vs baseline: 1.0588x; 1.0278x over previous
import numpy as np

import jax
import jax.numpy as jnp
from jax import lax
from jax.experimental import pallas as pl
from jax.experimental.pallas import tpu as pltpu
from jax.experimental.pallas import tpu_sc as plsc

N = 10000
F = 128
E = 320000
NP = 10240
NC, NS, L = 2, 16, 16
NW = NC * NS
CH = 128
EC = E // CH
FULL_CHUNKS = 80
LAST_CHUNKS = EC - (NW - 1) * FULL_CHUNKS
STRIPE = NP // NS
PACK = 16384
HC = 64
NEG = 5 * N
NEGC_PT = 16
NEGP = NW * NEGC_PT * CH

_mesh = lambda: plsc.VectorSubcoreMesh(
    core_axis_name="c", subcore_axis_name="s", num_cores=NC, num_subcores=NS)


def _tf2x32(k0, k1, c0, c1):
    rot1 = (13, 15, 26, 6)
    rot2 = (17, 29, 16, 24)
    ks = [np.uint32(k0), np.uint32(k1),
          np.uint32(k0) ^ np.uint32(k1) ^ np.uint32(0x1BD11BDA)]
    x0 = (c0 + ks[0]).astype(np.uint32)
    x1 = (c1 + ks[1]).astype(np.uint32)

    def rotl(x, d):
        return ((x << np.uint32(d)) | (x >> np.uint32(32 - d))).astype(np.uint32)

    for r in range(5):
        rots = rot1 if r % 2 == 0 else rot2
        for i in range(4):
            x0 = (x0 + x1).astype(np.uint32)
            x1 = rotl(x1, rots[i])
            x1 = (x1 ^ x0).astype(np.uint32)
        x0 = (x0 + ks[(r + 1) % 3]).astype(np.uint32)
        x1 = (x1 + ks[(r + 2) % 3] + np.uint32(r + 1)).astype(np.uint32)
    return x0, x1


def _tf_bits(k, n):
    idx = np.arange(n, dtype=np.uint32)
    y0, y1 = _tf2x32(k[0], k[1], np.zeros_like(idx), idx)
    return (y0 ^ y1).astype(np.uint32)


def _neg_pairs():
    idx = np.arange(2, dtype=np.uint32)
    y0, y1 = _tf2x32(0, 42, np.zeros_like(idx), idx)
    k1, k2 = (y0[0], y1[0]), (y0[1], y1[1])
    n = 2 * NEG
    hi = _tf_bits(k1, n)
    lo = _tf_bits(k2, n)
    span = np.uint32(N)
    mult = np.uint32((np.uint64(65536) % span) ** 2 % span)
    off = ((hi % span) * mult + lo % span) % span
    neg = off.astype(np.int32).reshape(2, NEG)
    padn = NEGP - NEG
    ar = np.arange(padn, dtype=np.int32)
    n0 = np.concatenate([neg[0], (N // 2) + (ar % (N // 2))])
    n1 = np.concatenate([neg[1], ar % (N // 2)])
    return (n0 * PACK + n1).astype(np.int32).reshape(NEGP // CH, CH)


_NEGC = _neg_pairs()


PCROWS = NW * FULL_CHUNKS


def _sc_deg_body(er_hbm, out_hbm, pc_hbm, r_v, c_v, z_v, hist_sh):
    cid = lax.axis_index("c")
    sid = lax.axis_index("s")
    wid = cid * NS + sid
    base = wid * FULL_CHUNKS

    @pl.when(wid < NW - 1)
    def _():
        pltpu.sync_copy(er_hbm.at[0, pl.ds(base, FULL_CHUNKS)], r_v)
        pltpu.sync_copy(er_hbm.at[1, pl.ds(base, FULL_CHUNKS)], c_v)

    @pl.when(wid == NW - 1)
    def _():
        pltpu.sync_copy(
            er_hbm.at[0, pl.ds((NW - 1) * FULL_CHUNKS, LAST_CHUNKS)],
            r_v.at[pl.ds(0, LAST_CHUNKS)])
        pltpu.sync_copy(
            er_hbm.at[1, pl.ds((NW - 1) * FULL_CHUNKS, LAST_CHUNKS)],
            c_v.at[pl.ds(0, LAST_CHUNKS)])
        lanes = lax.iota(jnp.int32, L)

        def fb(j, cc):
            for g in range(CH // L):
                x = lanes + g * L
                r_v[j, pl.ds(g * L, L)] = x
                c_v[j, pl.ds(g * L, L)] = N + lax.bitwise_and(x, 127)
            return cc
        lax.fori_loop(LAST_CHUNKS, FULL_CHUNKS, fb, 0)

    zeros16 = jnp.zeros((L,), jnp.float32)
    ones16 = jnp.ones((L,), jnp.float32)

    def zb(i, c):
        z_v[pl.ds(i * L, L)] = zeros16
        return c
    lax.fori_loop(0, STRIPE // L, zb, 0)
    pltpu.sync_copy(z_v, hist_sh.at[pl.ds(sid * STRIPE, STRIPE)])
    plsc.subcore_barrier()

    ones_v = z_v

    def ob(i, c):
        ones_v[pl.ds(i * L, L)] = ones16
        return c
    lax.fori_loop(0, CH // L, ob, 0)

    def sc(j, c):
        pltpu.sync_copy(ones_v.at[pl.ds(0, CH)], hist_sh.at[c_v.at[j]],
                        add=True)
        for g in range(CH // L):
            r = r_v[j, pl.ds(g * L, L)]
            cv = c_v[j, pl.ds(g * L, L)]
            r_v[j, pl.ds(g * L, L)] = r * PACK + cv
        return c
    lax.fori_loop(0, FULL_CHUNKS, sc, 0)
    pltpu.sync_copy(r_v, pc_hbm.at[pl.ds(base, FULL_CHUNKS)])
    plsc.subcore_barrier()
    pltpu.sync_copy(hist_sh.at[pl.ds(sid * STRIPE, STRIPE)],
                    out_hbm.at[cid, pl.ds(sid * STRIPE, STRIPE)])


_sc_deg = pl.kernel(
    _sc_deg_body,
    out_type=(jax.ShapeDtypeStruct((NC, NP), jnp.float32),
              jax.ShapeDtypeStruct((PCROWS, CH), jnp.int32)),
    mesh=_mesh(),
    scratch_types=[
        pltpu.VMEM((FULL_CHUNKS, CH), jnp.int32),
        pltpu.VMEM((FULL_CHUNKS, CH), jnp.int32),
        pltpu.VMEM((STRIPE,), jnp.float32),
        pltpu.VMEM_SHARED((NP,), jnp.float32),
    ],
)


def _su_stripe_init(su_hbm, dst_sh, sid):
    @pl.when(sid < NS - 1)
    def _():
        pltpu.sync_copy(su_hbm.at[pl.ds(sid * STRIPE, STRIPE)],
                        dst_sh.at[pl.ds(sid * STRIPE, STRIPE)])

    @pl.when(sid == NS - 1)
    def _():
        pltpu.sync_copy(su_hbm.at[pl.ds((NS - 1) * STRIPE, N - (NS - 1) * STRIPE)],
                        dst_sh.at[pl.ds((NS - 1) * STRIPE, N - (NS - 1) * STRIPE)])


def _sc_push_body(su_hbm, pc_hbm, out_hbm,
                  pidx_v, ridx_v, cidx_v, rows_v, acc_sh, gs0, gs1):
    cid = lax.axis_index("c")
    sid = lax.axis_index("s")
    wid = cid * NS + sid
    gsem = [gs0, gs1]
    pltpu.sync_copy(pc_hbm.at[pl.ds(wid * FULL_CHUNKS, FULL_CHUNKS)], pidx_v)
    _su_stripe_init(su_hbm, acc_sh, sid)
    plsc.subcore_barrier()

    def unpack(j, t):
        for g in range(CH // L):
            v = pidx_v[j, pl.ds(g * L, L)]
            ridx_v[t, pl.ds(g * L, L)] = lax.shift_right_logical(v, 14)
            cidx_v[t, pl.ds(g * L, L)] = lax.bitwise_and(v, PACK - 1)

    def gstart(t):
        pltpu.async_copy(su_hbm.at[ridx_v.at[t]], rows_v.at[t], gsem[t])

    def gwait(t):
        pltpu.make_async_copy(su_hbm.at[ridx_v.at[t]], rows_v.at[t],
                              gsem[t]).wait()

    unpack(0, 0)
    gstart(0)

    def body(k, c):
        for t in range(2):
            gwait(t)

            @pl.when(2 * k + t + 1 < FULL_CHUNKS)
            def _():
                unpack(2 * k + t + 1, 1 - t)
                gstart(1 - t)
            pltpu.sync_copy(rows_v.at[t], acc_sh.at[cidx_v.at[t]], add=True)
        return c
    lax.fori_loop(0, FULL_CHUNKS // 2, body, 0)
    plsc.subcore_barrier()
    pltpu.sync_copy(acc_sh.at[pl.ds(sid * STRIPE, STRIPE)],
                    out_hbm.at[cid, pl.ds(sid * STRIPE, STRIPE)])


_sc_push = pl.kernel(
    _sc_push_body,
    out_type=jax.ShapeDtypeStruct((NC, NP, F), jnp.float32),
    mesh=_mesh(),
    scratch_types=[
        pltpu.VMEM((FULL_CHUNKS, CH), jnp.int32),
        pltpu.VMEM((2, CH), jnp.int32),
        pltpu.VMEM((2, CH), jnp.int32),
        pltpu.VMEM((2, CH, F), jnp.float32),
        pltpu.VMEM_SHARED((NP, F), jnp.float32),
        pltpu.SemaphoreType.DMA,
        pltpu.SemaphoreType.DMA,
    ],
)


def _sc_loss_body(rep_hbm, pc_hbm, ng_hbm, out_hbm,
                  pidx_v, nidx_v, i0a_v, i1a_v, i0b_v, i1b_v,
                  rows0_v, rows1_v, acc_v, rep_sh,
                  g0A, g1A, g0B, g1B):
    cid = lax.axis_index("c")
    sid = lax.axis_index("s")
    wid = cid * NS + sid
    pltpu.sync_copy(pc_hbm.at[pl.ds(wid * FULL_CHUNKS, FULL_CHUNKS)], pidx_v)
    pltpu.sync_copy(ng_hbm.at[pl.ds(wid * NEGC_PT, NEGC_PT)], nidx_v)
    _su_stripe_init(rep_hbm, rep_sh, sid)
    plsc.subcore_barrier()
    lanes = lax.iota(jnp.int32, L)
    perms = [(lanes + k) % L for k in (8, 4, 2, 1)]

    def hsum(p):
        for pm in perms:
            p = p + jnp.take(p, pm)
        return p

    def mk_pv(idx_ref):
        def pv(h, g):
            j = lax.shift_right_logical(h, 1)
            base = lax.bitwise_and(h, 1) * HC
            return idx_ref[j, pl.ds(base + g * L, L)]
        return pv

    def mk_ops(pv):
        def unpack(h, i0buf, i1buf):
            for g in range(HC // L):
                v = pv(h, g)
                i0buf[pl.ds(g * L, L)] = lax.shift_right_logical(v, 14)
                i1buf[pl.ds(g * L, L)] = lax.bitwise_and(v, PACK - 1)

        def start(h, b):
            if b == 0:
                unpack(h, i0a_v, i1a_v)
                pltpu.async_copy(rep_sh.at[i0a_v], rows0_v.at[0], g0A)
                pltpu.async_copy(rep_sh.at[i1a_v], rows1_v.at[0], g1A)
            else:
                unpack(h, i0b_v, i1b_v)
                pltpu.async_copy(rep_sh.at[i0b_v], rows0_v.at[1], g0B)
                pltpu.async_copy(rep_sh.at[i1b_v], rows1_v.at[1], g1B)

        def wait(b):
            if b == 0:
                pltpu.make_async_copy(rep_sh.at[i0a_v], rows0_v.at[0], g0A).wait()
                pltpu.make_async_copy(rep_sh.at[i1a_v], rows1_v.at[0], g1A).wait()
            else:
                pltpu.make_async_copy(rep_sh.at[i0b_v], rows0_v.at[1], g0B).wait()
                pltpu.make_async_copy(rep_sh.at[i1b_v], rows1_v.at[1], g1B).wait()

        def chunk(h, b, tgt, carry):
            loss_a, cnt_a = carry
            rb0 = rows0_v.at[b]
            rb1 = rows1_v.at[b]
            for g in range(HC // L):
                v = pv(h, g)
                i0 = lax.shift_right_logical(v, 14)
                i1 = lax.bitwise_and(v, PACK - 1)
                msk = (i0 < i1) & (i1 < jnp.int32(N))

                def ebody(u2, dv):
                    for du in range(2):
                        u = 2 * u2 + du
                        e = g * L + u
                        p = jnp.zeros((L,), jnp.float32)
                        for k in range(F // L):
                            p = p + (rb0[e, pl.ds(k * L, L)]
                                     * rb1[e, pl.ds(k * L, L)])
                        s = hsum(p)
                        dv = jnp.where(lanes == u, s, dv)
                    return dv
                dot = lax.fori_loop(0, L // 2, ebody,
                                    jnp.zeros((L,), jnp.float32))
                d = dot - tgt
                loss_a = loss_a + jnp.where(msk, d * d, 0.0)
                cnt_a = cnt_a + jnp.where(msk, 1.0, 0.0)
            return loss_a, cnt_a

        return start, wait, chunk

    def run_region(ops, nh, tgt_s, carry):
        start, wait, chunk = ops
        tgt = jnp.full((L,), tgt_s, jnp.float32)
        start(0, 0)

        def body(k, carry):
            hA = 2 * k
            hB = hA + 1
            start(hB, 1)
            wait(0)
            carry = chunk(hA, 0, tgt, carry)

            @pl.when(hB + 1 < nh)
            def _():
                start(hB + 1, 0)

            wait(1)
            carry = chunk(hB, 1, tgt, carry)
            return carry
        return lax.fori_loop(0, nh // 2, body, carry)

    z = jnp.zeros((L,), jnp.float32)
    carry = run_region(mk_ops(mk_pv(pidx_v)), 2 * FULL_CHUNKS, 1.0, (z, z))
    carry = run_region(mk_ops(mk_pv(nidx_v)), 2 * NEGC_PT, 0.0, carry)
    loss_a, cnt_a = carry
    acc_v[0, :] = loss_a
    acc_v[1, :] = cnt_a
    pltpu.sync_copy(acc_v, out_hbm.at[wid])


_sc_loss = pl.kernel(
    _sc_loss_body,
    out_type=jax.ShapeDtypeStruct((NW, 2, L), jnp.float32),
    mesh=_mesh(),
    scratch_types=[
        pltpu.VMEM((FULL_CHUNKS, CH), jnp.int32),
        pltpu.VMEM((NEGC_PT, CH), jnp.int32),
        pltpu.VMEM((HC,), jnp.int32),
        pltpu.VMEM((HC,), jnp.int32),
        pltpu.VMEM((HC,), jnp.int32),
        pltpu.VMEM((HC,), jnp.int32),
        pltpu.VMEM((2, HC, F), jnp.float32),
        pltpu.VMEM((2, HC, F), jnp.float32),
        pltpu.VMEM((2, L), jnp.float32),
        pltpu.VMEM_SHARED((NP, F), jnp.float32),
        pltpu.SemaphoreType.DMA,
        pltpu.SemaphoreType.DMA,
        pltpu.SemaphoreType.DMA,
        pltpu.SemaphoreType.DMA,
    ],
)


BLK = 2000


def _tc1_body(x_ref, w_ref, d0_ref, d1_ref, o_ref):
    dinv = lax.rsqrt(d0_ref[...] + d1_ref[...] + 1.0)
    o_ref[...] = jnp.dot(x_ref[...], w_ref[...],
                         preferred_element_type=jnp.float32) * dinv


_tc1 = pl.pallas_call(
    _tc1_body,
    grid=(N // BLK,),
    in_specs=[
        pl.BlockSpec((BLK, F), lambda i: (i, 0)),
        pl.BlockSpec((F, F), lambda i: (0, 0)),
        pl.BlockSpec((BLK, 1), lambda i: (i, 0)),
        pl.BlockSpec((BLK, 1), lambda i: (i, 0)),
    ],
    out_specs=pl.BlockSpec((BLK, F), lambda i: (i, 0)),
    out_shape=jax.ShapeDtypeStruct((N, F), jnp.float32),
)


def _tc2_body(a0_ref, a1_ref, su_ref, d0_ref, d1_ref, b1_ref, w_ref, o_ref):
    dinv = lax.rsqrt(d0_ref[...] + d1_ref[...] + 1.0)
    base = (a0_ref[0] + a1_ref[0] - su_ref[...]) * dinv
    h = jnp.maximum(base + b1_ref[...], 0.0)
    o_ref[...] = jnp.dot(h, w_ref[...],
                         preferred_element_type=jnp.float32) * dinv


_tc2 = pl.pallas_call(
    _tc2_body,
    grid=(N // BLK,),
    in_specs=[
        pl.BlockSpec((1, BLK, F), lambda i: (0, i, 0)),
        pl.BlockSpec((1, BLK, F), lambda i: (1, i, 0)),
        pl.BlockSpec((BLK, F), lambda i: (i, 0)),
        pl.BlockSpec((BLK, 1), lambda i: (i, 0)),
        pl.BlockSpec((BLK, 1), lambda i: (i, 0)),
        pl.BlockSpec((1, F), lambda i: (0, 0)),
        pl.BlockSpec((F, F), lambda i: (0, 0)),
    ],
    out_specs=pl.BlockSpec((BLK, F), lambda i: (i, 0)),
    out_shape=jax.ShapeDtypeStruct((N, F), jnp.float32),
)


def _tc3_body(a0_ref, a1_ref, su_ref, d0_ref, d1_ref, b2_ref, o_ref):
    dinv = lax.rsqrt(d0_ref[...] + d1_ref[...] + 1.0)
    o_ref[...] = (a0_ref[0] + a1_ref[0] - su_ref[...]) * dinv + b2_ref[...]


_tc3 = pl.pallas_call(
    _tc3_body,
    grid=(N // BLK,),
    in_specs=[
        pl.BlockSpec((1, BLK, F), lambda i: (0, i, 0)),
        pl.BlockSpec((1, BLK, F), lambda i: (1, i, 0)),
        pl.BlockSpec((BLK, F), lambda i: (i, 0)),
        pl.BlockSpec((BLK, 1), lambda i: (i, 0)),
        pl.BlockSpec((BLK, 1), lambda i: (i, 0)),
        pl.BlockSpec((1, F), lambda i: (0, 0)),
    ],
    out_specs=pl.BlockSpec((BLK, F), lambda i: (i, 0)),
    out_shape=jax.ShapeDtypeStruct((N, F), jnp.float32),
)


def kernel(edge_index, features, W1, b1, W2, b2):
    er = edge_index.reshape(2, EC, CH)
    negc = jnp.asarray(_NEGC)

    deg_parts, pc = _sc_deg(er)
    d0 = deg_parts[0].reshape(NP, 1)
    d1 = deg_parts[1].reshape(NP, 1)
    b1r = b1.reshape(1, F)
    b2r = b2.reshape(1, F)

    su1 = _tc1(features, W1, d0, d1)
    acc1 = _sc_push(su1, pc)
    su2 = _tc2(acc1, acc1, su1, d0, d1, b1r, W2)
    acc2 = _sc_push(su2, pc)
    rep = _tc3(acc2, acc2, su2, d0, d1, b2r)

    parts = _sc_loss(rep, pc, negc)
    loss_sum = jnp.sum(parts[:, 0, :])
    cnt = jnp.sum(parts[:, 1, :])
    rec_loss = loss_sum * N / cnt
    return rep, rec_loss
